# Initial kernel scaffold; baseline (speedup 1.0000x reference)
import jax
import jax.numpy as jnp
from jax.experimental import pallas as pl


def kernel(x, edge_index, bn_weight, bn_bias):
    raise NotImplementedError("write your pallas kernel here")


# speedup vs baseline: 11.6467x; 11.6467x over previous
import jax
import jax.numpy as jnp
from jax import lax
from jax.experimental import pallas as pl
from jax.experimental.pallas import tpu as pltpu
from jax.experimental.pallas import tpu_sc as plsc

N_USERS = 2000
N_NODES = 10000
D = 128
E = 320000
ALPHA = 0.25

NC = 2
NS = 16
LN = 16
NW = NC * NS
NF = D // LN

NP = 10240
PW = NP // NW
PT = NP // NS
EPW = E // NW
CH = 100
NCHK = EPW // CH


def _mesh():
  return plsc.VectorSubcoreMesh(
      core_axis_name="c", subcore_axis_name="s", num_cores=NC, num_subcores=NS)


def _wid():
  return lax.axis_index("s") * NC + lax.axis_index("c")


def _rsqrt16(v):
  i = lax.bitcast_convert_type(v, jnp.int32)
  i = jnp.int32(0x5F3759DF) - lax.shift_right_logical(i, 1)
  y = lax.bitcast_convert_type(i, jnp.float32)
  for _ in range(3):
    y = y * (1.5 - 0.5 * v * y * y)
  return y


def _k1_body(cols, degp, colv, degv):
  w = _wid()

  @pl.loop(0, NP // LN)
  def _zero(g):
    degv[pl.ds(g * LN, LN)] = jnp.zeros((LN,), jnp.float32)

  pltpu.sync_copy(cols.at[pl.ds(w * EPW, EPW)], colv)
  ones = jnp.ones((LN,), jnp.float32)

  @pl.loop(0, EPW // LN)
  def _acc(g):
    idx = colv[pl.ds(g * LN, LN)]
    plsc.addupdate_scatter(degv, [idx], ones)

  pltpu.sync_copy(degv, degp.at[w])


def _k1():
  return pl.kernel(
      _k1_body,
      out_type=jax.ShapeDtypeStruct((NW, NP), jnp.float32),
      mesh=_mesh(),
      compiler_params=pltpu.CompilerParams(needs_layout_passes=False, use_tc_tiling_on_sc=False),
      scratch_types=[
          pltpu.VMEM((EPW,), jnp.int32),
          pltpu.VMEM((NP,), jnp.float32),
      ],
  )


def _k2_body(degp, x, dis, g0, dsum, disv, xbuf):
  w = _wid()
  off = w * PW
  pltpu.sync_copy(degp.at[:, pl.ds(off, PW)], dsum)

  @pl.loop(0, PW // LN)
  def _dis(g):
    tot = jnp.zeros((LN,), jnp.float32)
    for p in range(NW):
      tot = tot + dsum[p, pl.ds(g * LN, LN)]
    y = _rsqrt16(jnp.maximum(tot, 1.0))
    disv[pl.ds(g * LN, LN)] = jnp.where(tot > 0.0, y, 0.0)

  pltpu.sync_copy(x.at[pl.ds(off, PW)], xbuf)

  @pl.loop(0, PW // LN)
  def _scale(g):
    dg = disv[pl.ds(g * LN, LN)]
    for k in range(LN):
      bv = jnp.full((LN,), dg[k])
      for f in range(NF):
        sl = pl.ds(f * LN, LN)
        xbuf[g * LN + k, sl] = xbuf[g * LN + k, sl] * bv

  pltpu.sync_copy(xbuf, g0.at[pl.ds(off, PW)])
  pltpu.sync_copy(disv, dis.at[pl.ds(off, PW)])


def _k2():
  return pl.kernel(
      _k2_body,
      out_type=(
          jax.ShapeDtypeStruct((NP,), jnp.float32),
          jax.ShapeDtypeStruct((NP, D), jnp.float32),
      ),
      mesh=_mesh(),
      compiler_params=pltpu.CompilerParams(needs_layout_passes=False, use_tc_tiling_on_sc=False),
      scratch_types=[
          pltpu.VMEM((NW, PW), jnp.float32),
          pltpu.VMEM((PW,), jnp.float32),
          pltpu.VMEM((PW, D), jnp.float32),
      ],
  )


ZR = 64
SB = 10
NSB = NCHK // SB


def _k3_body(g, rows3, cols3, p, rowv, colv, gbuf, zbuf, acc, sem):
  c = lax.axis_index("c")
  s = lax.axis_index("s")
  w = s * NC + c

  @pl.loop(0, ZR)
  def _zz(i):
    for f in range(NF):
      zbuf[i, pl.ds(f * LN, LN)] = jnp.zeros((LN,), jnp.float32)

  for r in range(PT // ZR):
    pltpu.sync_copy(zbuf, acc.at[pl.ds(s * PT + r * ZR, ZR)])

  plsc.subcore_barrier()

  @pl.loop(0, NSB)
  def _slab(js):
    pltpu.sync_copy(rows3.at[w, pl.ds(js * SB, SB)], rowv)
    pltpu.sync_copy(cols3.at[w, pl.ds(js * SB, SB)], colv)

    @pl.loop(0, SB)
    def _edges(j):
      pltpu.async_copy(g.at[rowv.at[j]], gbuf, sem).wait()
      pltpu.sync_copy(gbuf, acc.at[colv.at[j]], add=True)

  plsc.subcore_barrier()
  pltpu.sync_copy(acc.at[pl.ds(s * PT, PT)], p.at[c, pl.ds(s * PT, PT)])


def _k3():
  return pl.kernel(
      _k3_body,
      out_type=jax.ShapeDtypeStruct((NC, NP, D), jnp.float32),
      mesh=_mesh(),
      compiler_params=pltpu.CompilerParams(needs_layout_passes=False, use_tc_tiling_on_sc=False),
      scratch_types=[
          pltpu.VMEM((SB, CH), jnp.int32),
          pltpu.VMEM((SB, CH), jnp.int32),
          pltpu.VMEM((CH, D), jnp.float32),
          pltpu.VMEM((ZR, D), jnp.float32),
          pltpu.VMEM_SHARED((NP, D), jnp.float32),
          pltpu.SemaphoreType.DMA,
      ],
  )


def _k4_body(p, dis, gout, b0, b1, disv):
  w = _wid()
  off = w * PW
  pltpu.sync_copy(p.at[0, pl.ds(off, PW)], b0)
  pltpu.sync_copy(p.at[1, pl.ds(off, PW)], b1)
  pltpu.sync_copy(dis.at[pl.ds(off, PW)], disv)

  @pl.loop(0, PW // LN)
  def _comb(g):
    dg = disv[pl.ds(g * LN, LN)]
    dg2 = dg * dg
    for k in range(LN):
      d2 = jnp.full((LN,), dg2[k])
      i = g * LN + k
      for f in range(NF):
        sl = pl.ds(f * LN, LN)
        b0[i, sl] = (b0[i, sl] + b1[i, sl]) * d2

  pltpu.sync_copy(b0, gout.at[pl.ds(off, PW)])


def _k4():
  return pl.kernel(
      _k4_body,
      out_type=jax.ShapeDtypeStruct((NP, D), jnp.float32),
      mesh=_mesh(),
      compiler_params=pltpu.CompilerParams(needs_layout_passes=False, use_tc_tiling_on_sc=False),
      scratch_types=[
          pltpu.VMEM((PW, D), jnp.float32),
          pltpu.VMEM((PW, D), jnp.float32),
          pltpu.VMEM((PW,), jnp.float32),
      ],
  )


def _k5_body(x, p1, p2, p3, dis, outb, stats, accb, tmpb, disv, statv):
  w = _wid()
  off = w * PW
  sl_rows = pl.ds(off, PW)
  pltpu.sync_copy(p1.at[0, sl_rows], accb)
  for ref in (p1.at[1, sl_rows], p2.at[0, sl_rows], p2.at[1, sl_rows],
              p3.at[0, sl_rows], p3.at[1, sl_rows]):
    pltpu.sync_copy(ref, tmpb)

    @pl.loop(0, PW)
    def _add(i):
      for f in range(NF):
        sl = pl.ds(f * LN, LN)
        accb[i, sl] = accb[i, sl] + tmpb[i, sl]

  pltpu.sync_copy(x.at[sl_rows], tmpb)
  pltpu.sync_copy(dis.at[sl_rows], disv)

  zeros = tuple(jnp.zeros((LN,), jnp.float32) for _ in range(2 * NF))

  @pl.loop(0, PW // LN, init_carry=zeros)
  def _out(g, carry):
    dg = disv[pl.ds(g * LN, LN)] * ALPHA
    sums = list(carry[:NF])
    sqs = list(carry[NF:])
    for k in range(LN):
      dv = jnp.full((LN,), dg[k])
      i = g * LN + k
      for f in range(NF):
        sl = pl.ds(f * LN, LN)
        o = tmpb[i, sl] * ALPHA + accb[i, sl] * dv
        accb[i, sl] = o
        sums[f] = sums[f] + o
        sqs[f] = sqs[f] + o * o
    return tuple(sums) + tuple(sqs)

  for f in range(NF):
    statv[0, pl.ds(f * LN, LN)] = _out[f]
    statv[1, pl.ds(f * LN, LN)] = _out[NF + f]

  pltpu.sync_copy(accb, outb.at[sl_rows])
  pltpu.sync_copy(statv, stats.at[w])


def _k5():
  return pl.kernel(
      _k5_body,
      out_type=(
          jax.ShapeDtypeStruct((NP, D), jnp.float32),
          jax.ShapeDtypeStruct((NW, 2, D), jnp.float32),
      ),
      mesh=_mesh(),
      compiler_params=pltpu.CompilerParams(needs_layout_passes=False, use_tc_tiling_on_sc=False),
      scratch_types=[
          pltpu.VMEM((PW, D), jnp.float32),
          pltpu.VMEM((PW, D), jnp.float32),
          pltpu.VMEM((PW,), jnp.float32),
          pltpu.VMEM((2, D), jnp.float32),
      ],
  )


def _k6_body(outb, stats, bnw, bnb, y, statv, buf, wv, bv):
  w = _wid()
  off = w * PW
  pltpu.sync_copy(stats, statv)
  pltpu.sync_copy(bnw, wv)
  pltpu.sync_copy(bnb, bv)

  inv_n = jnp.float32(1.0 / N_NODES)
  scales = []
  shifts = []
  for f in range(NF):
    sl = pl.ds(f * LN, LN)
    ssum = jnp.zeros((LN,), jnp.float32)
    ssq = jnp.zeros((LN,), jnp.float32)
    for p in range(NW):
      ssum = ssum + statv[p, 0, sl]
      ssq = ssq + statv[p, 1, sl]
    mean = ssum * inv_n
    var = ssq * inv_n - mean * mean
    rstd = _rsqrt16(var + 1e-5)
    scale = wv[sl] * rstd
    scales.append(scale)
    shifts.append(bv[sl] - mean * scale)

  pltpu.sync_copy(outb.at[pl.ds(off, PW)], buf)

  @pl.loop(0, PW)
  def _norm(i):
    for f in range(NF):
      sl = pl.ds(f * LN, LN)
      buf[i, sl] = buf[i, sl] * scales[f] + shifts[f]

  pltpu.sync_copy(buf, y.at[pl.ds(off, PW)])


def _k6():
  return pl.kernel(
      _k6_body,
      out_type=jax.ShapeDtypeStruct((NP, D), jnp.float32),
      mesh=_mesh(),
      compiler_params=pltpu.CompilerParams(needs_layout_passes=False, use_tc_tiling_on_sc=False),
      scratch_types=[
          pltpu.VMEM((NW, 2, D), jnp.float32),
          pltpu.VMEM((PW, D), jnp.float32),
          pltpu.VMEM((D,), jnp.float32),
          pltpu.VMEM((D,), jnp.float32),
      ],
  )


def kernel(x, edge_index, bn_weight, bn_bias):
  x = x.astype(jnp.float32)
  xp = jnp.zeros((NP, D), jnp.float32).at[:N_NODES].set(x)
  rows3 = edge_index[0].reshape(NW, NCHK, CH)
  cols3 = edge_index[1].reshape(NW, NCHK, CH)
  cols_flat = edge_index[1]

  degp = _k1()(cols_flat)
  dis, g0 = _k2()(degp, xp)
  k3 = _k3()
  k4 = _k4()
  p1 = k3(g0, rows3, cols3)
  g1 = k4(p1, dis)
  p2 = k3(g1, rows3, cols3)
  g2 = k4(p2, dis)
  p3 = k3(g2, rows3, cols3)
  outb, stats = _k5()(xp, p1, p2, p3, dis)
  y = _k6()(outb, stats, bn_weight, bn_bias)
  return (y[:N_USERS], y[N_USERS:N_NODES])


# speedup vs baseline: 13.9437x; 1.1972x over previous
import jax
import jax.numpy as jnp
from jax import lax
from jax.experimental import pallas as pl
from jax.experimental.pallas import tpu as pltpu
from jax.experimental.pallas import tpu_sc as plsc

N_USERS = 2000
N_NODES = 10000
D = 128
E = 320000
ALPHA = 0.25

NC = 2
NS = 16
LN = 16
NW = NC * NS
NF = D // LN

NP = 10240
PW = NP // NW
PT = NP // NS
EPW = E // NW
CH = 100
NCHK = EPW // CH


def _mesh():
  return plsc.VectorSubcoreMesh(
      core_axis_name="c", subcore_axis_name="s", num_cores=NC, num_subcores=NS)


def _wid():
  return lax.axis_index("s") * NC + lax.axis_index("c")


def _rsqrt16(v):
  i = lax.bitcast_convert_type(v, jnp.int32)
  i = jnp.int32(0x5F3759DF) - lax.shift_right_logical(i, 1)
  y = lax.bitcast_convert_type(i, jnp.float32)
  for _ in range(3):
    y = y * (1.5 - 0.5 * v * y * y)
  return y


def _k1_body(cols, degp, colv, degv):
  w = _wid()

  @pl.loop(0, NP // LN)
  def _zero(g):
    degv[pl.ds(g * LN, LN)] = jnp.zeros((LN,), jnp.float32)

  pltpu.sync_copy(cols.at[pl.ds(w * EPW, EPW)], colv)
  ones = jnp.ones((LN,), jnp.float32)

  @pl.loop(0, EPW // LN)
  def _acc(g):
    idx = colv[pl.ds(g * LN, LN)]
    plsc.addupdate_scatter(degv, [idx], ones)

  pltpu.sync_copy(degv, degp.at[w])


def _k1():
  return pl.kernel(
      _k1_body,
      out_type=jax.ShapeDtypeStruct((NW, NP), jnp.float32),
      mesh=_mesh(),
      compiler_params=pltpu.CompilerParams(needs_layout_passes=False, use_tc_tiling_on_sc=False),
      scratch_types=[
          pltpu.VMEM((EPW,), jnp.int32),
          pltpu.VMEM((NP,), jnp.float32),
      ],
  )


def _k2_body(degp, x, dis, g0, dsum, disv, xbuf):
  w = _wid()
  off = w * PW
  pltpu.sync_copy(degp.at[:, pl.ds(off, PW)], dsum)

  @pl.loop(0, PW // LN)
  def _dis(g):
    tot = jnp.zeros((LN,), jnp.float32)
    for p in range(NW):
      tot = tot + dsum[p, pl.ds(g * LN, LN)]
    y = _rsqrt16(jnp.maximum(tot, 1.0))
    disv[pl.ds(g * LN, LN)] = jnp.where(tot > 0.0, y, 0.0)

  pltpu.sync_copy(x.at[pl.ds(off, PW)], xbuf)

  @pl.loop(0, PW // LN)
  def _scale(g):
    dg = disv[pl.ds(g * LN, LN)]
    for k in range(LN):
      bv = jnp.full((LN,), dg[k])
      for f in range(NF):
        sl = pl.ds(f * LN, LN)
        xbuf[g * LN + k, sl] = xbuf[g * LN + k, sl] * bv

  pltpu.sync_copy(xbuf, g0.at[pl.ds(off, PW)])
  pltpu.sync_copy(disv, dis.at[pl.ds(off, PW)])


def _k2():
  return pl.kernel(
      _k2_body,
      out_type=(
          jax.ShapeDtypeStruct((NP,), jnp.float32),
          jax.ShapeDtypeStruct((NP, D), jnp.float32),
      ),
      mesh=_mesh(),
      compiler_params=pltpu.CompilerParams(needs_layout_passes=False, use_tc_tiling_on_sc=False),
      scratch_types=[
          pltpu.VMEM((NW, PW), jnp.float32),
          pltpu.VMEM((PW,), jnp.float32),
          pltpu.VMEM((PW, D), jnp.float32),
      ],
  )


ZR = 64
SB = 20
NSB = NCHK // SB


def _k3_body(g, rows3, cols3, p, rowv, colv, gbufa, gbufb, zbuf, acc,
             sga, sgb, ssa, ssb):
  c = lax.axis_index("c")
  s = lax.axis_index("s")
  w = s * NC + c

  @pl.loop(0, ZR)
  def _zz(i):
    for f in range(NF):
      zbuf[i, pl.ds(f * LN, LN)] = jnp.zeros((LN,), jnp.float32)

  for r in range(PT // ZR):
    pltpu.sync_copy(zbuf, acc.at[pl.ds(s * PT + r * ZR, ZR)])

  plsc.subcore_barrier()

  @pl.loop(0, NSB)
  def _slab(js):
    pltpu.sync_copy(rows3.at[w, pl.ds(js * SB, SB)], rowv)
    pltpu.sync_copy(cols3.at[w, pl.ds(js * SB, SB)], colv)

    @pl.loop(0, SB // 2)
    def _edges(jp):
      j0 = jp * 2
      j1 = jp * 2 + 1
      da = pltpu.async_copy(g.at[rowv.at[j0]], gbufa, sga)
      db = pltpu.async_copy(g.at[rowv.at[j1]], gbufb, sgb)
      da.wait()
      sa = pltpu.async_copy(gbufa, acc.at[colv.at[j0]], ssa, add=True)
      db.wait()
      sb = pltpu.async_copy(gbufb, acc.at[colv.at[j1]], ssb, add=True)
      sa.wait()
      sb.wait()

  plsc.subcore_barrier()
  pltpu.sync_copy(acc.at[pl.ds(s * PT, PT)], p.at[c, pl.ds(s * PT, PT)])


def _k3():
  return pl.kernel(
      _k3_body,
      out_type=jax.ShapeDtypeStruct((NC, NP, D), jnp.float32),
      mesh=_mesh(),
      compiler_params=pltpu.CompilerParams(needs_layout_passes=False, use_tc_tiling_on_sc=False),
      scratch_types=[
          pltpu.VMEM((SB, CH), jnp.int32),
          pltpu.VMEM((SB, CH), jnp.int32),
          pltpu.VMEM((CH, D), jnp.float32),
          pltpu.VMEM((CH, D), jnp.float32),
          pltpu.VMEM((ZR, D), jnp.float32),
          pltpu.VMEM_SHARED((NP, D), jnp.float32),
          pltpu.SemaphoreType.DMA,
          pltpu.SemaphoreType.DMA,
          pltpu.SemaphoreType.DMA,
          pltpu.SemaphoreType.DMA,
      ],
  )


def _k4_body(p, dis, gout, b0, b1, disv):
  w = _wid()
  off = w * PW
  pltpu.sync_copy(p.at[0, pl.ds(off, PW)], b0)
  pltpu.sync_copy(p.at[1, pl.ds(off, PW)], b1)
  pltpu.sync_copy(dis.at[pl.ds(off, PW)], disv)

  @pl.loop(0, PW // LN)
  def _comb(g):
    dg = disv[pl.ds(g * LN, LN)]
    dg2 = dg * dg
    for k in range(LN):
      d2 = jnp.full((LN,), dg2[k])
      i = g * LN + k
      for f in range(NF):
        sl = pl.ds(f * LN, LN)
        b0[i, sl] = (b0[i, sl] + b1[i, sl]) * d2

  pltpu.sync_copy(b0, gout.at[pl.ds(off, PW)])


def _k4():
  return pl.kernel(
      _k4_body,
      out_type=jax.ShapeDtypeStruct((NP, D), jnp.float32),
      mesh=_mesh(),
      compiler_params=pltpu.CompilerParams(needs_layout_passes=False, use_tc_tiling_on_sc=False),
      scratch_types=[
          pltpu.VMEM((PW, D), jnp.float32),
          pltpu.VMEM((PW, D), jnp.float32),
          pltpu.VMEM((PW,), jnp.float32),
      ],
  )


def _k5_body(x, p1, p2, p3, dis, outb, stats, accb, tmpb, disv, statv):
  w = _wid()
  off = w * PW
  sl_rows = pl.ds(off, PW)
  pltpu.sync_copy(p1.at[0, sl_rows], accb)
  for ref in (p1.at[1, sl_rows], p2.at[0, sl_rows], p2.at[1, sl_rows],
              p3.at[0, sl_rows], p3.at[1, sl_rows]):
    pltpu.sync_copy(ref, tmpb)

    @pl.loop(0, PW)
    def _add(i):
      for f in range(NF):
        sl = pl.ds(f * LN, LN)
        accb[i, sl] = accb[i, sl] + tmpb[i, sl]

  pltpu.sync_copy(x.at[sl_rows], tmpb)
  pltpu.sync_copy(dis.at[sl_rows], disv)

  zeros = tuple(jnp.zeros((LN,), jnp.float32) for _ in range(2 * NF))

  @pl.loop(0, PW // LN, init_carry=zeros)
  def _out(g, carry):
    dg = disv[pl.ds(g * LN, LN)] * ALPHA
    sums = list(carry[:NF])
    sqs = list(carry[NF:])
    for k in range(LN):
      dv = jnp.full((LN,), dg[k])
      i = g * LN + k
      for f in range(NF):
        sl = pl.ds(f * LN, LN)
        o = tmpb[i, sl] * ALPHA + accb[i, sl] * dv
        accb[i, sl] = o
        sums[f] = sums[f] + o
        sqs[f] = sqs[f] + o * o
    return tuple(sums) + tuple(sqs)

  for f in range(NF):
    statv[0, pl.ds(f * LN, LN)] = _out[f]
    statv[1, pl.ds(f * LN, LN)] = _out[NF + f]

  pltpu.sync_copy(accb, outb.at[sl_rows])
  pltpu.sync_copy(statv, stats.at[w])


def _k5():
  return pl.kernel(
      _k5_body,
      out_type=(
          jax.ShapeDtypeStruct((NP, D), jnp.float32),
          jax.ShapeDtypeStruct((NW, 2, D), jnp.float32),
      ),
      mesh=_mesh(),
      compiler_params=pltpu.CompilerParams(needs_layout_passes=False, use_tc_tiling_on_sc=False),
      scratch_types=[
          pltpu.VMEM((PW, D), jnp.float32),
          pltpu.VMEM((PW, D), jnp.float32),
          pltpu.VMEM((PW,), jnp.float32),
          pltpu.VMEM((2, D), jnp.float32),
      ],
  )


def _k6_body(outb, stats, bnw, bnb, y, statv, buf, wv, bv):
  w = _wid()
  off = w * PW
  pltpu.sync_copy(stats, statv)
  pltpu.sync_copy(bnw, wv)
  pltpu.sync_copy(bnb, bv)

  inv_n = jnp.float32(1.0 / N_NODES)
  scales = []
  shifts = []
  for f in range(NF):
    sl = pl.ds(f * LN, LN)
    ssum = jnp.zeros((LN,), jnp.float32)
    ssq = jnp.zeros((LN,), jnp.float32)
    for p in range(NW):
      ssum = ssum + statv[p, 0, sl]
      ssq = ssq + statv[p, 1, sl]
    mean = ssum * inv_n
    var = ssq * inv_n - mean * mean
    rstd = _rsqrt16(var + 1e-5)
    scale = wv[sl] * rstd
    scales.append(scale)
    shifts.append(bv[sl] - mean * scale)

  pltpu.sync_copy(outb.at[pl.ds(off, PW)], buf)

  @pl.loop(0, PW)
  def _norm(i):
    for f in range(NF):
      sl = pl.ds(f * LN, LN)
      buf[i, sl] = buf[i, sl] * scales[f] + shifts[f]

  pltpu.sync_copy(buf, y.at[pl.ds(off, PW)])


def _k6():
  return pl.kernel(
      _k6_body,
      out_type=jax.ShapeDtypeStruct((NP, D), jnp.float32),
      mesh=_mesh(),
      compiler_params=pltpu.CompilerParams(needs_layout_passes=False, use_tc_tiling_on_sc=False),
      scratch_types=[
          pltpu.VMEM((NW, 2, D), jnp.float32),
          pltpu.VMEM((PW, D), jnp.float32),
          pltpu.VMEM((D,), jnp.float32),
          pltpu.VMEM((D,), jnp.float32),
      ],
  )


def kernel(x, edge_index, bn_weight, bn_bias):
  x = x.astype(jnp.float32)
  xp = jnp.zeros((NP, D), jnp.float32).at[:N_NODES].set(x)
  rows3 = edge_index[0].reshape(NW, NCHK, CH)
  cols3 = edge_index[1].reshape(NW, NCHK, CH)
  cols_flat = edge_index[1]

  degp = _k1()(cols_flat)
  dis, g0 = _k2()(degp, xp)
  k3 = _k3()
  k4 = _k4()
  p1 = k3(g0, rows3, cols3)
  g1 = k4(p1, dis)
  p2 = k3(g1, rows3, cols3)
  g2 = k4(p2, dis)
  p3 = k3(g2, rows3, cols3)
  outb, stats = _k5()(xp, p1, p2, p3, dis)
  y = _k6()(outb, stats, bn_weight, bn_bias)
  return (y[:N_USERS], y[N_USERS:N_NODES])


# speedup vs baseline: 16.4750x; 1.1815x over previous
import jax
import jax.numpy as jnp
from jax import lax
from jax.experimental import pallas as pl
from jax.experimental.pallas import tpu as pltpu
from jax.experimental.pallas import tpu_sc as plsc

N_USERS = 2000
N_NODES = 10000
D = 128
E = 320000
ALPHA = 0.25

NC = 2
NS = 16
LN = 16
NW = NC * NS
NF = D // LN

NP = 10240
PW = NP // NW
PT = NP // NS
EPW = E // NW
CH = 50
NCHK = EPW // CH


def _mesh():
  return plsc.VectorSubcoreMesh(
      core_axis_name="c", subcore_axis_name="s", num_cores=NC, num_subcores=NS)


def _wid():
  return lax.axis_index("s") * NC + lax.axis_index("c")


def _rsqrt16(v):
  i = lax.bitcast_convert_type(v, jnp.int32)
  i = jnp.int32(0x5F3759DF) - lax.shift_right_logical(i, 1)
  y = lax.bitcast_convert_type(i, jnp.float32)
  for _ in range(3):
    y = y * (1.5 - 0.5 * v * y * y)
  return y


def _k1_body(cols, degp, colv, degv):
  w = _wid()

  @pl.loop(0, NP // LN)
  def _zero(g):
    degv[pl.ds(g * LN, LN)] = jnp.zeros((LN,), jnp.float32)

  pltpu.sync_copy(cols.at[pl.ds(w * EPW, EPW)], colv)
  ones = jnp.ones((LN,), jnp.float32)

  @pl.loop(0, EPW // LN)
  def _acc(g):
    idx = colv[pl.ds(g * LN, LN)]
    plsc.addupdate_scatter(degv, [idx], ones)

  pltpu.sync_copy(degv, degp.at[w])


def _k1():
  return pl.kernel(
      _k1_body,
      out_type=jax.ShapeDtypeStruct((NW, NP), jnp.float32),
      mesh=_mesh(),
      compiler_params=pltpu.CompilerParams(needs_layout_passes=False, use_tc_tiling_on_sc=False),
      scratch_types=[
          pltpu.VMEM((EPW,), jnp.int32),
          pltpu.VMEM((NP,), jnp.float32),
      ],
  )


def _k2_body(degp, x, dis, g0, dsum, disv, xbuf):
  w = _wid()
  off = w * PW
  pltpu.sync_copy(degp.at[:, pl.ds(off, PW)], dsum)

  @pl.loop(0, PW // LN)
  def _dis(g):
    tot = jnp.zeros((LN,), jnp.float32)
    for p in range(NW):
      tot = tot + dsum[p, pl.ds(g * LN, LN)]
    y = _rsqrt16(jnp.maximum(tot, 1.0))
    disv[pl.ds(g * LN, LN)] = jnp.where(tot > 0.0, y, 0.0)

  pltpu.sync_copy(x.at[pl.ds(off, PW)], xbuf)

  @pl.loop(0, PW // LN)
  def _scale(g):
    dg = disv[pl.ds(g * LN, LN)]
    for k in range(LN):
      bv = jnp.full((LN,), dg[k])
      for f in range(NF):
        sl = pl.ds(f * LN, LN)
        xbuf[g * LN + k, sl] = xbuf[g * LN + k, sl] * bv

  pltpu.sync_copy(xbuf, g0.at[pl.ds(off, PW)])
  pltpu.sync_copy(disv, dis.at[pl.ds(off, PW)])


def _k2():
  return pl.kernel(
      _k2_body,
      out_type=(
          jax.ShapeDtypeStruct((NP,), jnp.float32),
          jax.ShapeDtypeStruct((NP, D), jnp.float32),
      ),
      mesh=_mesh(),
      compiler_params=pltpu.CompilerParams(needs_layout_passes=False, use_tc_tiling_on_sc=False),
      scratch_types=[
          pltpu.VMEM((NW, PW), jnp.float32),
          pltpu.VMEM((PW,), jnp.float32),
          pltpu.VMEM((PW, D), jnp.float32),
      ],
  )


ZR = 64
SB = 40
NSB = NCHK // SB
NB = 5
NGRP = SB // NB


def _k3_body(g, rows3, cols3, p, rowv, colv, zbuf, acc, *bufs_and_sems):
  gbufs = bufs_and_sems[:NB]
  sg = bufs_and_sems[NB:2 * NB]
  ss = bufs_and_sems[2 * NB:3 * NB]
  c = lax.axis_index("c")
  s = lax.axis_index("s")
  w = s * NC + c

  @pl.loop(0, ZR)
  def _zz(i):
    for f in range(NF):
      zbuf[i, pl.ds(f * LN, LN)] = jnp.zeros((LN,), jnp.float32)

  for r in range(PT // ZR):
    pltpu.sync_copy(zbuf, acc.at[pl.ds(s * PT + r * ZR, ZR)])

  plsc.subcore_barrier()

  def _wait_gather(b, j):
    pltpu.make_async_copy(g.at[rowv.at[j]], gbufs[b], sg[b]).wait()

  def _wait_scatter(b, j):
    pltpu.make_async_copy(gbufs[b], acc.at[colv.at[j]], ss[b]).wait()

  @pl.loop(0, NSB)
  def _slab(js):
    pltpu.sync_copy(rows3.at[w, pl.ds(js * SB, SB)], rowv)
    pltpu.sync_copy(cols3.at[w, pl.ds(js * SB, SB)], colv)

    for b in range(NB):
      pltpu.async_copy(g.at[rowv.at[b]], gbufs[b], sg[b])

    @pl.loop(0, NGRP - 1)
    def _grp(gi):
      base = gi * NB
      for b in range(NB):
        _wait_gather(b, base + b)
        pltpu.async_copy(gbufs[b], acc.at[colv.at[base + b]], ss[b], add=True)
      for b in range(NB):
        _wait_scatter(b, base + b)
        pltpu.async_copy(g.at[rowv.at[base + NB + b]], gbufs[b], sg[b])

    last = (NGRP - 1) * NB
    for b in range(NB):
      _wait_gather(b, last + b)
      pltpu.async_copy(gbufs[b], acc.at[colv.at[last + b]], ss[b], add=True)
    for b in range(NB):
      _wait_scatter(b, last + b)

  plsc.subcore_barrier()
  pltpu.sync_copy(acc.at[pl.ds(s * PT, PT)], p.at[c, pl.ds(s * PT, PT)])


def _k3():
  return pl.kernel(
      _k3_body,
      out_type=jax.ShapeDtypeStruct((NC, NP, D), jnp.float32),
      mesh=_mesh(),
      compiler_params=pltpu.CompilerParams(needs_layout_passes=False, use_tc_tiling_on_sc=False),
      scratch_types=[
          pltpu.VMEM((SB, CH), jnp.int32),
          pltpu.VMEM((SB, CH), jnp.int32),
          pltpu.VMEM((ZR, D), jnp.float32),
          pltpu.VMEM_SHARED((NP, D), jnp.float32),
      ] + [pltpu.VMEM((CH, D), jnp.float32) for _ in range(NB)]
        + [pltpu.SemaphoreType.DMA for _ in range(2 * NB)],
  )


def _k4_body(p, dis, gout, b0, b1, disv):
  w = _wid()
  off = w * PW
  pltpu.sync_copy(p.at[0, pl.ds(off, PW)], b0)
  pltpu.sync_copy(p.at[1, pl.ds(off, PW)], b1)
  pltpu.sync_copy(dis.at[pl.ds(off, PW)], disv)

  @pl.loop(0, PW // LN)
  def _comb(g):
    dg = disv[pl.ds(g * LN, LN)]
    dg2 = dg * dg
    for k in range(LN):
      d2 = jnp.full((LN,), dg2[k])
      i = g * LN + k
      for f in range(NF):
        sl = pl.ds(f * LN, LN)
        b0[i, sl] = (b0[i, sl] + b1[i, sl]) * d2

  pltpu.sync_copy(b0, gout.at[pl.ds(off, PW)])


def _k4():
  return pl.kernel(
      _k4_body,
      out_type=jax.ShapeDtypeStruct((NP, D), jnp.float32),
      mesh=_mesh(),
      compiler_params=pltpu.CompilerParams(needs_layout_passes=False, use_tc_tiling_on_sc=False),
      scratch_types=[
          pltpu.VMEM((PW, D), jnp.float32),
          pltpu.VMEM((PW, D), jnp.float32),
          pltpu.VMEM((PW,), jnp.float32),
      ],
  )


def _k5_body(x, p1, p2, p3, dis, outb, stats, accb, tmpb, disv, statv):
  w = _wid()
  off = w * PW
  sl_rows = pl.ds(off, PW)
  pltpu.sync_copy(p1.at[0, sl_rows], accb)
  for ref in (p1.at[1, sl_rows], p2.at[0, sl_rows], p2.at[1, sl_rows],
              p3.at[0, sl_rows], p3.at[1, sl_rows]):
    pltpu.sync_copy(ref, tmpb)

    @pl.loop(0, PW)
    def _add(i):
      for f in range(NF):
        sl = pl.ds(f * LN, LN)
        accb[i, sl] = accb[i, sl] + tmpb[i, sl]

  pltpu.sync_copy(x.at[sl_rows], tmpb)
  pltpu.sync_copy(dis.at[sl_rows], disv)

  zeros = tuple(jnp.zeros((LN,), jnp.float32) for _ in range(2 * NF))

  @pl.loop(0, PW // LN, init_carry=zeros)
  def _out(g, carry):
    dg = disv[pl.ds(g * LN, LN)] * ALPHA
    sums = list(carry[:NF])
    sqs = list(carry[NF:])
    for k in range(LN):
      dv = jnp.full((LN,), dg[k])
      i = g * LN + k
      for f in range(NF):
        sl = pl.ds(f * LN, LN)
        o = tmpb[i, sl] * ALPHA + accb[i, sl] * dv
        accb[i, sl] = o
        sums[f] = sums[f] + o
        sqs[f] = sqs[f] + o * o
    return tuple(sums) + tuple(sqs)

  for f in range(NF):
    statv[0, pl.ds(f * LN, LN)] = _out[f]
    statv[1, pl.ds(f * LN, LN)] = _out[NF + f]

  pltpu.sync_copy(accb, outb.at[sl_rows])
  pltpu.sync_copy(statv, stats.at[w])


def _k5():
  return pl.kernel(
      _k5_body,
      out_type=(
          jax.ShapeDtypeStruct((NP, D), jnp.float32),
          jax.ShapeDtypeStruct((NW, 2, D), jnp.float32),
      ),
      mesh=_mesh(),
      compiler_params=pltpu.CompilerParams(needs_layout_passes=False, use_tc_tiling_on_sc=False),
      scratch_types=[
          pltpu.VMEM((PW, D), jnp.float32),
          pltpu.VMEM((PW, D), jnp.float32),
          pltpu.VMEM((PW,), jnp.float32),
          pltpu.VMEM((2, D), jnp.float32),
      ],
  )


def _k6_body(outb, stats, bnw, bnb, y, statv, buf, wv, bv):
  w = _wid()
  off = w * PW
  pltpu.sync_copy(stats, statv)
  pltpu.sync_copy(bnw, wv)
  pltpu.sync_copy(bnb, bv)

  inv_n = jnp.float32(1.0 / N_NODES)
  scales = []
  shifts = []
  for f in range(NF):
    sl = pl.ds(f * LN, LN)
    ssum = jnp.zeros((LN,), jnp.float32)
    ssq = jnp.zeros((LN,), jnp.float32)
    for p in range(NW):
      ssum = ssum + statv[p, 0, sl]
      ssq = ssq + statv[p, 1, sl]
    mean = ssum * inv_n
    var = ssq * inv_n - mean * mean
    rstd = _rsqrt16(var + 1e-5)
    scale = wv[sl] * rstd
    scales.append(scale)
    shifts.append(bv[sl] - mean * scale)

  pltpu.sync_copy(outb.at[pl.ds(off, PW)], buf)

  @pl.loop(0, PW)
  def _norm(i):
    for f in range(NF):
      sl = pl.ds(f * LN, LN)
      buf[i, sl] = buf[i, sl] * scales[f] + shifts[f]

  pltpu.sync_copy(buf, y.at[pl.ds(off, PW)])


def _k6():
  return pl.kernel(
      _k6_body,
      out_type=jax.ShapeDtypeStruct((NP, D), jnp.float32),
      mesh=_mesh(),
      compiler_params=pltpu.CompilerParams(needs_layout_passes=False, use_tc_tiling_on_sc=False),
      scratch_types=[
          pltpu.VMEM((NW, 2, D), jnp.float32),
          pltpu.VMEM((PW, D), jnp.float32),
          pltpu.VMEM((D,), jnp.float32),
          pltpu.VMEM((D,), jnp.float32),
      ],
  )


def kernel(x, edge_index, bn_weight, bn_bias):
  x = x.astype(jnp.float32)
  xp = jnp.zeros((NP, D), jnp.float32).at[:N_NODES].set(x)
  rows3 = edge_index[0].reshape(NW, NCHK, CH)
  cols3 = edge_index[1].reshape(NW, NCHK, CH)
  cols_flat = edge_index[1]

  degp = _k1()(cols_flat)
  dis, g0 = _k2()(degp, xp)
  k3 = _k3()
  k4 = _k4()
  p1 = k3(g0, rows3, cols3)
  g1 = k4(p1, dis)
  p2 = k3(g1, rows3, cols3)
  g2 = k4(p2, dis)
  p3 = k3(g2, rows3, cols3)
  outb, stats = _k5()(xp, p1, p2, p3, dis)
  y = _k6()(outb, stats, bn_weight, bn_bias)
  return (y[:N_USERS], y[N_USERS:N_NODES])
